# Initial kernel scaffold; baseline (speedup 1.0000x reference)
import jax
import jax.numpy as jnp
from jax.experimental import pallas as pl


def kernel(features, image, conv1_w, conv1_b, reg_w, reg_b, cls_w, cls_b, eye_w, eye_b):
    raise NotImplementedError("write your pallas kernel here")


# speedup vs baseline: 25.4112x; 25.4112x over previous
import functools

import numpy as np
import jax
import jax.numpy as jnp
from jax.experimental import pallas as pl
from jax.experimental.pallas import tpu as pltpu

_NEG_INF = float("-inf")


def _anchor_constants(fh, fw, ih, iw):
    xs = float(iw) / float(fw)
    ys = float(ih) / float(fh)
    x_centers = np.arange(xs / 2.0, iw, xs, dtype=np.float32)
    y_centers = np.arange(ys / 2.0, ih, ys, dtype=np.float32)
    xc, yc = np.meshgrid(x_centers, y_centers, indexing='xy')
    centers = np.stack([xc.reshape(-1), yc.reshape(-1)], axis=-1)
    ratios = np.array([0.5, 1.0, 2.0], dtype=np.float32)
    scales = np.array([8.0, 16.0, 32.0], dtype=np.float32)
    s, r = np.meshgrid(scales, ratios, indexing='xy')
    s = s.reshape(-1)
    r = r.reshape(-1)
    heights = (np.sqrt(s ** 2 / r) * ys).reshape(-1)
    widths = (heights * r * xs / ys).reshape(-1)
    nc = centers.shape[0]
    na = heights.size
    centers = np.tile(centers[:, None, :], (1, na, 1))
    heights = np.tile(heights[None, :], (nc, 1))
    widths = np.tile(widths[None, :], (nc, 1))
    x_min = centers[:, :, 0] - widths / 2.0
    y_min = centers[:, :, 1] - heights / 2.0
    x_max = centers[:, :, 0] + widths / 2.0
    y_max = centers[:, :, 1] + heights / 2.0
    anchors = np.stack([x_min, y_min, x_max, y_max], axis=-1).reshape(-1, 4).astype(np.float32)
    mask = ((anchors[:, 0] >= 0.0) & (anchors[:, 1] >= 0.0)
            & (anchors[:, 2] <= float(iw)) & (anchors[:, 3] <= float(ih)))
    return anchors, mask


def _nms_kernel(ncell, na, max_out, ih, iw, inside0,
                xpad_ref, w9_ref, b1_ref, wd_ref, bd_ref, we_ref, be_ref,
                anc_ref, ins_ref, lin_ref,
                eye_out_ref, sel_out_ref,
                act_s, h_s, x1_s, y1_s, x2_s, y2_s, ar_s, s_s):
    f32 = jnp.float32
    batch = act_s.shape[0] // ncell
    fh = fw = int(np.sqrt(ncell))

    for b in range(batch):
        acc = None
        for k in range(9):
            ky, kx = k // 3, k % 3
            sl = xpad_ref[b, ky:ky + fh, kx:kx + fw, :].reshape(ncell, -1)
            p = jnp.dot(sl, w9_ref[k], preferred_element_type=f32)
            acc = p if acc is None else acc + p
        act_s[b * ncell:(b + 1) * ncell, :] = jnp.maximum(acc + b1_ref[:], 0.0)

    act = act_s[:]
    eye_out_ref[:] = jnp.dot(act, we_ref[:], preferred_element_type=f32) + be_ref[:]
    ht = jnp.dot(act, wd_ref[:], preferred_element_type=f32) + bd_ref[:]
    h_s[:] = jnp.swapaxes(ht, 0, 1)

    acx = anc_ref[0 * na:1 * na, :]
    acy = anc_ref[1 * na:2 * na, :]
    aw = anc_ref[2 * na:3 * na, :]
    ah = anc_ref[3 * na:4 * na, :]
    for b in range(batch):
        cs = slice(b * ncell, (b + 1) * ncell)
        d0 = h_s[0 * na:1 * na, cs]
        d1 = h_s[1 * na:2 * na, cs]
        d2 = h_s[2 * na:3 * na, cs]
        d3 = h_s[3 * na:4 * na, cs]
        logit = h_s[4 * na:5 * na, cs]
        cx = d0 * aw + acx
        cy = d1 * ah + acy
        w = jnp.exp(d2) * aw
        h = jnp.exp(d3) * ah
        x1 = jnp.clip(cx - w / 2.0, 0.0, float(iw))
        y1 = jnp.clip(cy - h / 2.0, 0.0, float(ih))
        x2 = jnp.clip(cx + w / 2.0, 0.0, float(iw))
        y2 = jnp.clip(cy + h / 2.0, 0.0, float(ih))
        x1_s[:, cs] = x1
        y1_s[:, cs] = y1
        x2_s[:, cs] = x2
        y2_s[:, cs] = y2
        ar_s[:, cs] = (x2 - x1) * (y2 - y1)
        s_s[:, cs] = jax.nn.sigmoid(logit) + ins_ref[:]

    lin = lin_ref[:]
    lane8 = jax.lax.broadcasted_iota(jnp.int32, (1, 8), 1)

    def it_body(t, carry):
        for b in range(batch):
            cs = slice(b * ncell, (b + 1) * ncell)
            s = s_s[:, cs]
            m = jnp.max(s)
            cand = jnp.where(s == m, lin, jnp.int32(2 ** 30))
            io = jnp.min(cand)
            io = jnp.where(m == -jnp.inf, jnp.int32(inside0), io)
            msk = lin == io

            def g(a):
                return jnp.sum(jnp.where(msk, a, 0.0))

            x1v = x1_s[:, cs]
            y1v = y1_s[:, cs]
            x2v = x2_s[:, cs]
            y2v = y2_s[:, cs]
            bx1 = g(x1v)
            by1 = g(y1v)
            bx2 = g(x2v)
            by2 = g(y2v)
            q0 = g(h_s[0 * na:1 * na, cs])
            q1 = g(h_s[1 * na:2 * na, cs])
            q2 = g(h_s[2 * na:3 * na, cs])
            q3 = g(h_s[3 * na:4 * na, cs])
            ai = (bx2 - bx1) * (by2 - by1)
            xx1 = jnp.maximum(bx1, x1v)
            yy1 = jnp.maximum(by1, y1v)
            xx2 = jnp.minimum(bx2, x2v)
            yy2 = jnp.minimum(by2, y2v)
            inter = jnp.maximum(xx2 - xx1, 0.0) * jnp.maximum(yy2 - yy1, 0.0)
            iou = inter / (ai + ar_s[:, cs] - inter + 1e-9)
            kill = (iou > 0.5) | msk
            s_s[:, cs] = jnp.where(kill, -jnp.inf, s)

            vals = ((bx1 + bx2) / 2.0, (by1 + by2) / 2.0, bx2 - bx1, by2 - by1,
                    q0, q1, q2, q3)
            row = jnp.zeros((1, 8), f32)
            for k, v in enumerate(vals):
                row = jnp.where(lane8 == k, v, row)
            sel_out_ref[pl.ds(b * max_out + t, 1), :] = row
        return carry

    jax.lax.fori_loop(0, max_out, it_body, 0)


def kernel(features, image, conv1_w, conv1_b, reg_w, reg_b, cls_w, cls_b, eye_w, eye_b):
    f32 = jnp.float32
    B, fh, fw, C = features.shape
    ih, iw = image.shape[1], image.shape[2]
    ncell = fh * fw
    na = 9
    max_out = 256

    anchors, mask = _anchor_constants(fh, fw, ih, iw)
    inside_idx = np.nonzero(mask)[0]
    inside0 = int(inside_idx[0])

    a4 = anchors.reshape(ncell, na, 4).transpose(1, 0, 2)
    acx = (a4[:, :, 0] + a4[:, :, 2]) / 2.0
    acy = (a4[:, :, 1] + a4[:, :, 3]) / 2.0
    aww = a4[:, :, 2] - a4[:, :, 0]
    ahh = a4[:, :, 3] - a4[:, :, 1]
    anc = np.concatenate([acx, acy, aww, ahh], axis=0).astype(np.float32)
    mT = mask.reshape(ncell, na).T
    ins = np.where(mT, np.float32(0.0), np.float32(_NEG_INF))
    lin = np.ascontiguousarray(
        np.arange(ncell * na, dtype=np.int32).reshape(ncell, na).T)

    xpad = jnp.pad(features, ((0, 0), (1, 1), (1, 1), (0, 0)))
    w9 = conv1_w.reshape(9, C, C)
    b1 = conv1_b.reshape(1, C)
    rw = reg_w.reshape(C, na, 4).transpose(0, 2, 1).reshape(C, 4 * na)
    cw = cls_w.reshape(C, na)
    wd = jnp.concatenate([rw, cw, jnp.zeros((C, 128 - 5 * na), f32)], axis=1)
    rb = reg_b.reshape(na, 4).transpose(1, 0).reshape(4 * na)
    bd = jnp.concatenate([rb, cls_b, jnp.zeros((128 - 5 * na,), f32)]).reshape(1, 128)
    we = eye_w.reshape(C, 24)
    be = eye_b.reshape(1, 24)

    body = functools.partial(_nms_kernel, ncell, na, max_out, ih, iw, inside0)
    eye_out, sel = pl.pallas_call(
        body,
        out_shape=[
            jax.ShapeDtypeStruct((B * ncell, 24), f32),
            jax.ShapeDtypeStruct((B * max_out, 8), f32),
        ],
        scratch_shapes=[
            pltpu.VMEM((B * ncell, C), f32),
            pltpu.VMEM((128, B * ncell), f32),
            pltpu.VMEM((na, B * ncell), f32),
            pltpu.VMEM((na, B * ncell), f32),
            pltpu.VMEM((na, B * ncell), f32),
            pltpu.VMEM((na, B * ncell), f32),
            pltpu.VMEM((na, B * ncell), f32),
            pltpu.VMEM((na, B * ncell), f32),
        ],
    )(xpad, w9, b1, wd, bd, we, be,
      jnp.asarray(anc), jnp.asarray(ins), jnp.asarray(lin))

    rois = sel[:, 0:4].reshape(B, max_out, 4)
    preds = sel[:, 4:8].reshape(B, max_out, 4)
    anchors_b = jnp.tile(jnp.asarray(anchors[mask])[None, :, :], (B, 1, 1))
    return rois, eye_out, preds, anchors_b


# speedup vs baseline: 26.4063x; 1.0392x over previous
import functools

import numpy as np
import jax
import jax.numpy as jnp
from jax.experimental import pallas as pl
from jax.experimental.pallas import tpu as pltpu

_NEG_INF = float("-inf")


def _anchor_constants(fh, fw, ih, iw):
    xs = float(iw) / float(fw)
    ys = float(ih) / float(fh)
    x_centers = np.arange(xs / 2.0, iw, xs, dtype=np.float32)
    y_centers = np.arange(ys / 2.0, ih, ys, dtype=np.float32)
    xc, yc = np.meshgrid(x_centers, y_centers, indexing='xy')
    centers = np.stack([xc.reshape(-1), yc.reshape(-1)], axis=-1)
    ratios = np.array([0.5, 1.0, 2.0], dtype=np.float32)
    scales = np.array([8.0, 16.0, 32.0], dtype=np.float32)
    s, r = np.meshgrid(scales, ratios, indexing='xy')
    s = s.reshape(-1)
    r = r.reshape(-1)
    heights = (np.sqrt(s ** 2 / r) * ys).reshape(-1)
    widths = (heights * r * xs / ys).reshape(-1)
    nc = centers.shape[0]
    na = heights.size
    centers = np.tile(centers[:, None, :], (1, na, 1))
    heights = np.tile(heights[None, :], (nc, 1))
    widths = np.tile(widths[None, :], (nc, 1))
    x_min = centers[:, :, 0] - widths / 2.0
    y_min = centers[:, :, 1] - heights / 2.0
    x_max = centers[:, :, 0] + widths / 2.0
    y_max = centers[:, :, 1] + heights / 2.0
    anchors = np.stack([x_min, y_min, x_max, y_max], axis=-1).reshape(-1, 4).astype(np.float32)
    mask = ((anchors[:, 0] >= 0.0) & (anchors[:, 1] >= 0.0)
            & (anchors[:, 2] <= float(iw)) & (anchors[:, 3] <= float(ih)))
    return anchors, mask


def _nms_kernel(ncell, na, max_out, ih, iw, inside0,
                xpad_ref, w9_ref, b1_ref, wd_ref, bd_ref, we_ref, be_ref,
                anc_ref, ins_ref, lin_ref,
                eye_out_ref, sel_out_ref,
                act_s, h_s, x1_s, y1_s, x2_s, y2_s, ar_s, s_s,
                d0_s, d1_s, d2_s, d3_s):
    f32 = jnp.float32
    batch = act_s.shape[0] // ncell
    fh = fw = int(np.sqrt(ncell))
    nrow = ncell * na // 128

    for b in range(batch):
        acc = None
        for k in range(9):
            ky, kx = k // 3, k % 3
            sl = xpad_ref[b, ky:ky + fh, kx:kx + fw, :].reshape(ncell, -1)
            p = jnp.dot(sl, w9_ref[k], preferred_element_type=f32)
            acc = p if acc is None else acc + p
        act_s[b * ncell:(b + 1) * ncell, :] = jnp.maximum(acc + b1_ref[:], 0.0)

    act = act_s[:]
    eye_out_ref[:] = jnp.dot(act, we_ref[:], preferred_element_type=f32) + be_ref[:]
    ht = jnp.dot(act, wd_ref[:], preferred_element_type=f32) + bd_ref[:]
    h_s[:] = jnp.swapaxes(ht, 0, 1)

    acx = anc_ref[0 * na:1 * na, :]
    acy = anc_ref[1 * na:2 * na, :]
    aw = anc_ref[2 * na:3 * na, :]
    ah = anc_ref[3 * na:4 * na, :]
    for b in range(batch):
        cs = slice(b * ncell, (b + 1) * ncell)
        rs = slice(b * nrow, (b + 1) * nrow)
        d0 = h_s[0 * na:1 * na, cs]
        d1 = h_s[1 * na:2 * na, cs]
        d2 = h_s[2 * na:3 * na, cs]
        d3 = h_s[3 * na:4 * na, cs]
        logit = h_s[4 * na:5 * na, cs]
        cx = d0 * aw + acx
        cy = d1 * ah + acy
        w = jnp.exp(d2) * aw
        h = jnp.exp(d3) * ah
        x1 = jnp.clip(cx - w / 2.0, 0.0, float(iw))
        y1 = jnp.clip(cy - h / 2.0, 0.0, float(ih))
        x2 = jnp.clip(cx + w / 2.0, 0.0, float(iw))
        y2 = jnp.clip(cy + h / 2.0, 0.0, float(ih))
        d0_s[rs, :] = d0.reshape(nrow, 128)
        d1_s[rs, :] = d1.reshape(nrow, 128)
        d2_s[rs, :] = d2.reshape(nrow, 128)
        d3_s[rs, :] = d3.reshape(nrow, 128)
        x1_s[rs, :] = x1.reshape(nrow, 128)
        y1_s[rs, :] = y1.reshape(nrow, 128)
        x2_s[rs, :] = x2.reshape(nrow, 128)
        y2_s[rs, :] = y2.reshape(nrow, 128)
        ar_s[rs, :] = ((x2 - x1) * (y2 - y1)).reshape(nrow, 128)
        s_s[rs, :] = jax.nn.sigmoid(logit).reshape(nrow, 128) + ins_ref[:]

    lin = lin_ref[:]
    lane8 = jax.lax.broadcasted_iota(jnp.int32, (1, 8), 1)
    lane128 = jax.lax.broadcasted_iota(jnp.int32, (1, 128), 1)

    def it_body(t, carry):
        for b in range(batch):
            rs = slice(b * nrow, (b + 1) * nrow)
            s = s_s[rs, :]
            m = jnp.max(s)
            cand = jnp.where(s == m, lin, jnp.int32(2 ** 30))
            io = jnp.min(cand)
            io = jnp.where(m == -jnp.inf, jnp.int32(inside0), io)
            cc = io // na
            aa = io - cc * na
            rr = b * nrow + aa * (ncell // 128) + cc // 128
            ll = cc % 128
            lmask = lane128 == ll

            def g(ref):
                row = ref[pl.ds(rr, 1), :]
                return jnp.sum(jnp.where(lmask, row, 0.0), axis=1, keepdims=True)

            bx1 = g(x1_s)
            by1 = g(y1_s)
            bx2 = g(x2_s)
            by2 = g(y2_s)
            ai = g(ar_s)
            q0 = g(d0_s)
            q1 = g(d1_s)
            q2 = g(d2_s)
            q3 = g(d3_s)
            x1v = x1_s[rs, :]
            y1v = y1_s[rs, :]
            x2v = x2_s[rs, :]
            y2v = y2_s[rs, :]
            xx1 = jnp.maximum(bx1, x1v)
            yy1 = jnp.maximum(by1, y1v)
            xx2 = jnp.minimum(bx2, x2v)
            yy2 = jnp.minimum(by2, y2v)
            inter = jnp.maximum(xx2 - xx1, 0.0) * jnp.maximum(yy2 - yy1, 0.0)
            iou = inter / (ai + ar_s[rs, :] - inter + 1e-9)
            s_s[rs, :] = jnp.where(iou > 0.5, -jnp.inf, s)
            srow = s_s[pl.ds(rr, 1), :]
            s_s[pl.ds(rr, 1), :] = jnp.where(lmask, -jnp.inf, srow)

            vals = ((bx1 + bx2) / 2.0, (by1 + by2) / 2.0, bx2 - bx1, by2 - by1,
                    q0, q1, q2, q3)
            row = jnp.zeros((1, 8), f32)
            for k, v in enumerate(vals):
                row = jnp.where(lane8 == k, v, row)
            sel_out_ref[pl.ds(b * max_out + t, 1), :] = row
        return carry

    jax.lax.fori_loop(0, max_out, it_body, 0)


def kernel(features, image, conv1_w, conv1_b, reg_w, reg_b, cls_w, cls_b, eye_w, eye_b):
    f32 = jnp.float32
    B, fh, fw, C = features.shape
    ih, iw = image.shape[1], image.shape[2]
    ncell = fh * fw
    na = 9
    max_out = 256

    anchors, mask = _anchor_constants(fh, fw, ih, iw)
    inside_idx = np.nonzero(mask)[0]
    inside0 = int(inside_idx[0])

    a4 = anchors.reshape(ncell, na, 4).transpose(1, 0, 2)
    acx = (a4[:, :, 0] + a4[:, :, 2]) / 2.0
    acy = (a4[:, :, 1] + a4[:, :, 3]) / 2.0
    aww = a4[:, :, 2] - a4[:, :, 0]
    ahh = a4[:, :, 3] - a4[:, :, 1]
    anc = np.concatenate([acx, acy, aww, ahh], axis=0).astype(np.float32)
    nrow = ncell * na // 128
    mT = mask.reshape(ncell, na).T
    ins = np.where(mT, np.float32(0.0), np.float32(_NEG_INF)).reshape(nrow, 128)
    lin = np.ascontiguousarray(
        np.arange(ncell * na, dtype=np.int32).reshape(ncell, na).T).reshape(nrow, 128)

    xpad = jnp.pad(features, ((0, 0), (1, 1), (1, 1), (0, 0)))
    w9 = conv1_w.reshape(9, C, C)
    b1 = conv1_b.reshape(1, C)
    rw = reg_w.reshape(C, na, 4).transpose(0, 2, 1).reshape(C, 4 * na)
    cw = cls_w.reshape(C, na)
    wd = jnp.concatenate([rw, cw, jnp.zeros((C, 128 - 5 * na), f32)], axis=1)
    rb = reg_b.reshape(na, 4).transpose(1, 0).reshape(4 * na)
    bd = jnp.concatenate([rb, cls_b, jnp.zeros((128 - 5 * na,), f32)]).reshape(1, 128)
    we = eye_w.reshape(C, 24)
    be = eye_b.reshape(1, 24)

    body = functools.partial(_nms_kernel, ncell, na, max_out, ih, iw, inside0)
    eye_out, sel = pl.pallas_call(
        body,
        out_shape=[
            jax.ShapeDtypeStruct((B * ncell, 24), f32),
            jax.ShapeDtypeStruct((B * max_out, 8), f32),
        ],
        scratch_shapes=[
            pltpu.VMEM((B * ncell, C), f32),
            pltpu.VMEM((128, B * ncell), f32),
            pltpu.VMEM((B * nrow, 128), f32),
            pltpu.VMEM((B * nrow, 128), f32),
            pltpu.VMEM((B * nrow, 128), f32),
            pltpu.VMEM((B * nrow, 128), f32),
            pltpu.VMEM((B * nrow, 128), f32),
            pltpu.VMEM((B * nrow, 128), f32),
            pltpu.VMEM((B * nrow, 128), f32),
            pltpu.VMEM((B * nrow, 128), f32),
            pltpu.VMEM((B * nrow, 128), f32),
            pltpu.VMEM((B * nrow, 128), f32),
        ],
    )(xpad, w9, b1, wd, bd, we, be,
      jnp.asarray(anc), jnp.asarray(ins), jnp.asarray(lin))

    rois = sel[:, 0:4].reshape(B, max_out, 4)
    preds = sel[:, 4:8].reshape(B, max_out, 4)
    anchors_b = jnp.tile(jnp.asarray(anchors[mask])[None, :, :], (B, 1, 1))
    return rois, eye_out, preds, anchors_b


# speedup vs baseline: 41.0182x; 1.5533x over previous
import functools

import numpy as np
import jax
import jax.numpy as jnp
from jax.experimental import pallas as pl
from jax.experimental.pallas import tpu as pltpu

_NEG_INF = float("-inf")


def _anchor_constants(fh, fw, ih, iw):
    xs = float(iw) / float(fw)
    ys = float(ih) / float(fh)
    x_centers = np.arange(xs / 2.0, iw, xs, dtype=np.float32)
    y_centers = np.arange(ys / 2.0, ih, ys, dtype=np.float32)
    xc, yc = np.meshgrid(x_centers, y_centers, indexing='xy')
    centers = np.stack([xc.reshape(-1), yc.reshape(-1)], axis=-1)
    ratios = np.array([0.5, 1.0, 2.0], dtype=np.float32)
    scales = np.array([8.0, 16.0, 32.0], dtype=np.float32)
    s, r = np.meshgrid(scales, ratios, indexing='xy')
    s = s.reshape(-1)
    r = r.reshape(-1)
    heights = (np.sqrt(s ** 2 / r) * ys).reshape(-1)
    widths = (heights * r * xs / ys).reshape(-1)
    nc = centers.shape[0]
    na = heights.size
    centers = np.tile(centers[:, None, :], (1, na, 1))
    heights = np.tile(heights[None, :], (nc, 1))
    widths = np.tile(widths[None, :], (nc, 1))
    x_min = centers[:, :, 0] - widths / 2.0
    y_min = centers[:, :, 1] - heights / 2.0
    x_max = centers[:, :, 0] + widths / 2.0
    y_max = centers[:, :, 1] + heights / 2.0
    anchors = np.stack([x_min, y_min, x_max, y_max], axis=-1).reshape(-1, 4).astype(np.float32)
    mask = ((anchors[:, 0] >= 0.0) & (anchors[:, 1] >= 0.0)
            & (anchors[:, 2] <= float(iw)) & (anchors[:, 3] <= float(ih)))
    return anchors, mask


def _nms_kernel(ncell, na, max_out, ih, iw, inside0,
                xs0_ref, xs1_ref, xs2_ref, w9_ref, b1_ref, wd_ref, bd_ref,
                we_ref, be_ref, anc_ref, ins_ref, lin_ref,
                eye_out_ref, sel_out_ref,
                act_s, h_s, *box_refs):
    xs_refs = (xs0_ref, xs1_ref, xs2_ref)
    f32 = jnp.float32
    batch = act_s.shape[0] // ncell
    fh = fw = int(np.sqrt(ncell))
    nrow = ncell * na // 128
    brefs = [box_refs[9 * b:9 * (b + 1)] for b in range(batch)]

    for b in range(batch):
        acc = None
        for k in range(9):
            ky, kx = k // 3, k % 3
            sl = xs_refs[kx][b, ky:ky + fh, :, :].reshape(ncell, -1)
            p = jnp.dot(sl, w9_ref[k], preferred_element_type=f32)
            acc = p if acc is None else acc + p
        act_s[b * ncell:(b + 1) * ncell, :] = jnp.maximum(acc + b1_ref[:], 0.0)

    act = act_s[:]
    eye_out_ref[:] = jnp.dot(act, we_ref[:], preferred_element_type=f32) + be_ref[:]
    ht = jnp.dot(act, wd_ref[:], preferred_element_type=f32) + bd_ref[:]
    h_s[:] = jnp.swapaxes(ht, 0, 1)

    acx = anc_ref[0 * na:1 * na, :]
    acy = anc_ref[1 * na:2 * na, :]
    aw = anc_ref[2 * na:3 * na, :]
    ah = anc_ref[3 * na:4 * na, :]
    s_init = []
    for b in range(batch):
        cs = slice(b * ncell, (b + 1) * ncell)
        x1_s, y1_s, x2_s, y2_s, ar_s, d0_s, d1_s, d2_s, d3_s = brefs[b]
        d0 = h_s[0 * na:1 * na, cs]
        d1 = h_s[1 * na:2 * na, cs]
        d2 = h_s[2 * na:3 * na, cs]
        d3 = h_s[3 * na:4 * na, cs]
        logit = h_s[4 * na:5 * na, cs]
        cx = d0 * aw + acx
        cy = d1 * ah + acy
        w = jnp.exp(d2) * aw
        h = jnp.exp(d3) * ah
        x1 = jnp.clip(cx - w / 2.0, 0.0, float(iw))
        y1 = jnp.clip(cy - h / 2.0, 0.0, float(ih))
        x2 = jnp.clip(cx + w / 2.0, 0.0, float(iw))
        y2 = jnp.clip(cy + h / 2.0, 0.0, float(ih))
        d0_s[...] = d0.reshape(nrow, 128)
        d1_s[...] = d1.reshape(nrow, 128)
        d2_s[...] = d2.reshape(nrow, 128)
        d3_s[...] = d3.reshape(nrow, 128)
        x1_s[...] = x1.reshape(nrow, 128)
        y1_s[...] = y1.reshape(nrow, 128)
        x2_s[...] = x2.reshape(nrow, 128)
        y2_s[...] = y2.reshape(nrow, 128)
        ar_s[...] = ((x2 - x1) * (y2 - y1)).reshape(nrow, 128)
        s_init.append(jax.nn.sigmoid(logit).reshape(nrow, 128) + ins_ref[:])

    lin = lin_ref[:]
    lane8 = jax.lax.broadcasted_iota(jnp.int32, (1, 8), 1)
    inf = jnp.float32(jnp.inf)
    ax01 = (0, 1)
    pad_i = jnp.full((1, 1), inside0, jnp.int32)

    def it_body(t, ss):
        tabs = [[r[...] for r in brefs[b]] for b in range(batch)]
        ms = [jnp.max(ss[b], axis=ax01, keepdims=True) for b in range(batch)]
        cands = [jnp.where(ss[b] == ms[b], lin, jnp.int32(2 ** 30))
                 for b in range(batch)]
        ios = [jnp.min(cands[b], axis=ax01, keepdims=True) for b in range(batch)]
        ios = [jnp.where(ms[b] == -jnp.inf, pad_i, ios[b])
               for b in range(batch)]
        msks = [lin == ios[b] for b in range(batch)]
        gath = [[jnp.min(jnp.where(msks[b], a, inf), axis=ax01, keepdims=True)
                 for a in tabs[b]] for b in range(batch)]
        new_ss = []
        for b in range(batch):
            bx1, by1, bx2, by2, ai, q0, q1, q2, q3 = gath[b]
            x1v, y1v, x2v, y2v, arv = tabs[b][:5]
            xx1 = jnp.maximum(bx1, x1v)
            yy1 = jnp.maximum(by1, y1v)
            xx2 = jnp.minimum(bx2, x2v)
            yy2 = jnp.minimum(by2, y2v)
            inter = jnp.maximum(xx2 - xx1, 0.0) * jnp.maximum(yy2 - yy1, 0.0)
            iou = inter / (ai + arv - inter + 1e-9)
            kill = (iou > 0.5) | msks[b]
            new_ss.append(jnp.where(kill, -jnp.inf, ss[b]))

            vals = ((bx1 + bx2) / 2.0, (by1 + by2) / 2.0, bx2 - bx1, by2 - by1,
                    q0, q1, q2, q3)
            row = jnp.zeros((1, 8), f32)
            for k, v in enumerate(vals):
                row = jnp.where(lane8 == k, v, row)
            sel_out_ref[pl.ds(b * max_out + t, 1), :] = row
        return tuple(new_ss)

    jax.lax.fori_loop(0, max_out, it_body, tuple(s_init))


def kernel(features, image, conv1_w, conv1_b, reg_w, reg_b, cls_w, cls_b, eye_w, eye_b):
    f32 = jnp.float32
    B, fh, fw, C = features.shape
    ih, iw = image.shape[1], image.shape[2]
    ncell = fh * fw
    na = 9
    max_out = 256

    anchors, mask = _anchor_constants(fh, fw, ih, iw)
    inside_idx = np.nonzero(mask)[0]
    inside0 = int(inside_idx[0])

    a4 = anchors.reshape(ncell, na, 4).transpose(1, 0, 2)
    acx = (a4[:, :, 0] + a4[:, :, 2]) / 2.0
    acy = (a4[:, :, 1] + a4[:, :, 3]) / 2.0
    aww = a4[:, :, 2] - a4[:, :, 0]
    ahh = a4[:, :, 3] - a4[:, :, 1]
    anc = np.concatenate([acx, acy, aww, ahh], axis=0).astype(np.float32)
    nrow = ncell * na // 128
    mT = mask.reshape(ncell, na).T
    ins = np.where(mT, np.float32(0.0), np.float32(_NEG_INF)).reshape(nrow, 128)
    lin = np.ascontiguousarray(
        np.arange(ncell * na, dtype=np.int32).reshape(ncell, na).T).reshape(nrow, 128)

    xpad = jnp.pad(features, ((0, 0), (1, 1), (1, 1), (0, 0)))
    xs = [xpad[:, :, dx:dx + fw, :] for dx in range(3)]
    w9 = conv1_w.reshape(9, C, C)
    b1 = conv1_b.reshape(1, C)
    rw = reg_w.reshape(C, na, 4).transpose(0, 2, 1).reshape(C, 4 * na)
    cw = cls_w.reshape(C, na)
    wd = jnp.concatenate([rw, cw, jnp.zeros((C, 128 - 5 * na), f32)], axis=1)
    rb = reg_b.reshape(na, 4).transpose(1, 0).reshape(4 * na)
    bd = jnp.concatenate([rb, cls_b, jnp.zeros((128 - 5 * na,), f32)]).reshape(1, 128)
    we = eye_w.reshape(C, 24)
    be = eye_b.reshape(1, 24)

    body = functools.partial(_nms_kernel, ncell, na, max_out, ih, iw, inside0)
    eye_out, sel = pl.pallas_call(
        body,
        out_shape=[
            jax.ShapeDtypeStruct((B * ncell, 24), f32),
            jax.ShapeDtypeStruct((B * max_out, 8), f32),
        ],
        scratch_shapes=(
            [pltpu.VMEM((B * ncell, C), f32),
             pltpu.VMEM((128, B * ncell), f32)]
            + [pltpu.VMEM((nrow, 128), f32) for _ in range(9 * B)]
        ),
    )(xs[0], xs[1], xs[2], w9, b1, wd, bd, we, be,
      jnp.asarray(anc), jnp.asarray(ins), jnp.asarray(lin))

    rois = sel[:, 0:4].reshape(B, max_out, 4)
    preds = sel[:, 4:8].reshape(B, max_out, 4)
    anchors_b = jnp.tile(jnp.asarray(anchors[mask])[None, :, :], (B, 1, 1))
    return rois, eye_out, preds, anchors_b
